# baseline (speedup 1.0000x reference)
import functools

import jax
import jax.numpy as jnp
from jax import lax
from jax.experimental import pallas as pl
from jax.experimental.pallas import tpu as pltpu
from jax.experimental.pallas import tpu_sc as plsc

B, L, V, D = 4096, 50, 1000000, 64
NC, NS = 2, 16
NW = NC * NS
BPW = B // NW
CB = 2
NCHUNK = BPW // CB
CIDX = CB * L
LANES = 16


def _sc_kernel(idx_hbm, table_hbm, out_hbm, idx_v, buf, out_v, sem):
    wid = lax.axis_index("s") * NC + lax.axis_index("c")
    pltpu.sync_copy(idx_hbm.at[pl.ds(wid * NCHUNK, NCHUNK)], idx_v)

    def chunk_body(c, carry):
        pltpu.async_copy(table_hbm.at[idx_v.at[c]], buf, sem).wait()
        for ro in range(CB):
            base = ro * L
            for q in range(D // LANES):
                acc = buf[base, pl.ds(q * LANES, LANES)]
                for j in range(1, L):
                    acc = acc + buf[base + j, pl.ds(q * LANES, LANES)]
                out_v[c * CB + ro, pl.ds(q * LANES, LANES)] = acc
        return carry

    lax.fori_loop(0, NCHUNK, chunk_body, 0)
    pltpu.sync_copy(out_v, out_hbm.at[pl.ds(wid * BPW, BPW)])


@jax.jit
def _run(idx2d, table):
    mesh = plsc.VectorSubcoreMesh(
        core_axis_name="c", subcore_axis_name="s",
        num_cores=NC, num_subcores=NS)
    return pl.kernel(
        _sc_kernel,
        out_type=jax.ShapeDtypeStruct((B, D), jnp.float32),
        mesh=mesh,
        scratch_types=[
            pltpu.VMEM((NCHUNK, CIDX), jnp.int32),
            pltpu.VMEM((CIDX, D), jnp.float32),
            pltpu.VMEM((BPW, D), jnp.float32),
            pltpu.SemaphoreType.DMA,
        ],
        compiler_params=pltpu.CompilerParams(use_tc_tiling_on_sc=False),
    )(idx2d, table)


def kernel(idx, table):
    idx2d = idx.astype(jnp.int32).reshape(NW * NCHUNK, CIDX)
    out = _run(idx2d, table)
    return out[:, None, :]


# speedup vs baseline: 1.0627x; 1.0627x over previous
import functools

import jax
import jax.numpy as jnp
from jax import lax
from jax.experimental import pallas as pl
from jax.experimental.pallas import tpu as pltpu
from jax.experimental.pallas import tpu_sc as plsc

B, L, V, D = 4096, 50, 1000000, 64
NC, NS = 2, 16
NW = NC * NS
BPW = B // NW
CB = 2
NCHUNK = BPW // CB
CIDX = CB * L
LANES = 16


NBUF = 4
KCH = 4


def _sc_kernel(idx_hbm, table_hbm, out_hbm, idx_v, bufs, out_v, *sems):
    wid = lax.axis_index("s") * NC + lax.axis_index("c")
    pltpu.sync_copy(idx_hbm.at[pl.ds(wid * NCHUNK, NCHUNK)], idx_v)

    def issue(c, b):
        pltpu.async_copy(table_hbm.at[idx_v.at[c]], bufs.at[b], sems[b])

    for b in range(NBUF):
        issue(b, b)

    def group_body(g, carry):
        for b in range(NBUF):
            c = g * NBUF + b
            pltpu.make_async_copy(
                table_hbm.at[idx_v.at[c]], bufs.at[b], sems[b]).wait()
            buf = bufs.at[b]
            for ro in range(CB):
                base = ro * L
                for q in range(D // LANES):
                    ds = pl.ds(q * LANES, LANES)
                    accs = [None] * KCH
                    for j in range(L):
                        v = buf[base + j, ds]
                        k = j % KCH
                        accs[k] = v if accs[k] is None else accs[k] + v
                    while len(accs) > 1:
                        accs = [a + bb for a, bb in zip(accs[::2], accs[1::2])] \
                            + ([accs[-1]] if len(accs) % 2 else [])
                    out_v[c * CB + ro, ds] = accs[0]
            nxt = c + NBUF

            @pl.when(nxt < NCHUNK)
            def _():
                issue(nxt, b)
        return carry

    lax.fori_loop(0, NCHUNK // NBUF, group_body, 0)
    pltpu.sync_copy(out_v, out_hbm.at[pl.ds(wid * BPW, BPW)])


@jax.jit
def _run(idx2d, table):
    mesh = plsc.VectorSubcoreMesh(
        core_axis_name="c", subcore_axis_name="s",
        num_cores=NC, num_subcores=NS)
    return pl.kernel(
        _sc_kernel,
        out_type=jax.ShapeDtypeStruct((B, D), jnp.float32),
        mesh=mesh,
        scratch_types=[
            pltpu.VMEM((NCHUNK, CIDX), jnp.int32),
            pltpu.VMEM((NBUF, CIDX, D), jnp.float32),
            pltpu.VMEM((BPW, D), jnp.float32),
        ] + [pltpu.SemaphoreType.DMA] * NBUF,
        compiler_params=pltpu.CompilerParams(use_tc_tiling_on_sc=False),
    )(idx2d, table)


def kernel(idx, table):
    idx2d = idx.astype(jnp.int32).reshape(NW * NCHUNK, CIDX)
    out = _run(idx2d, table)
    return out[:, None, :]
